# baseline (speedup 1.0000x reference)
import functools

import jax
import jax.numpy as jnp
from jax import lax
from jax.experimental import pallas as pl
from jax.experimental.pallas import tpu as pltpu
from jax.experimental.pallas import tpu_sc as plsc

D_MODEL = 128
MAX_LEN = 200
BATCH = 1024
TABLE_ROWS = MAX_LEN + 1

B_TC = 512
B_SC = BATCH - B_TC
BB = 128

NUM_CORES = 2
NUM_SUBCORES = 16
NW = NUM_CORES * NUM_SUBCORES
B_PER_W = BATCH // NW
SB = B_SC // NW
ROWS_PER_W = B_PER_W * MAX_LEN
LANES = 16
GROUPS_PER_PAIR = 2 * MAX_LEN // LANES
Q = 25
NQ = MAX_LEN // Q
QE = Q * D_MODEL


def _full(v):
    return jnp.full((LANES,), v, dtype=jnp.int32)


@functools.partial(
    pl.kernel,
    out_type=(
        jax.ShapeDtypeStruct((BATCH * MAX_LEN * D_MODEL,), jnp.float32),
        jax.ShapeDtypeStruct((BATCH * MAX_LEN,), jnp.int32),
    ),
    mesh=plsc.VectorSubcoreMesh(core_axis_name="c", subcore_axis_name="s"),
    scratch_types=[
        pltpu.VMEM((B_PER_W,), jnp.int32),
        pltpu.VMEM((SB,), jnp.int32),
        pltpu.SMEM((SB,), jnp.int32),
        pltpu.VMEM((ROWS_PER_W,), jnp.int32),
        pltpu.VMEM((TABLE_ROWS * D_MODEL,), jnp.float32),
        pltpu.VMEM((QE,), jnp.float32),
        pltpu.SemaphoreType.DMA,
        pltpu.SemaphoreType.DMA,
    ],
)
def _pe_fill_sc(pe_hbm, len_hbm, pos_out, idx_out,
                len_v, len2_v, len2_s, idx_v, table_f, zero_f, wsem, isem):
    wid = lax.axis_index("s") * NUM_CORES + lax.axis_index("c")
    base_b = wid * B_PER_W
    base_r = wid * ROWS_PER_W
    pos_b0 = B_TC + wid * SB

    pltpu.sync_copy(len_hbm.at[pl.ds(base_b, B_PER_W)], len_v)
    pltpu.sync_copy(len_hbm.at[pl.ds(pos_b0, SB)], len2_v)
    pltpu.sync_copy(pe_hbm, table_f)

    for h in range(SB // LANES):
        lens16_s = len2_v[pl.ds(h * LANES, LANES)]
        for t in range(LANES):
            len2_s[h * LANES + t] = lens16_s[t]

    lanes = lax.iota(jnp.int32, LANES)
    fz = jnp.full((LANES,), 0.0, dtype=jnp.float32)

    def zfill(i, carry):
        zero_f[pl.ds(i * LANES, LANES)] = fz
        return carry

    lax.fori_loop(0, QE // LANES, zfill, 0)

    def do_batch(b, carry):
        blen = len2_s[b]
        obase = pl.multiple_of((pos_b0 + b) * MAX_LEN * D_MODEL, QE)
        for q in range(NQ):
            qs = q * Q
            dst = pos_out.at[pl.ds(obase + qs * D_MODEL, QE)]

            @pl.when(blen >= qs + Q)
            def _(dst=dst, qs=qs):
                pltpu.async_copy(
                    table_f.at[pl.ds((1 + qs) * D_MODEL, QE)], dst, wsem)

            @pl.when(blen <= qs)
            def _(dst=dst):
                pltpu.async_copy(zero_f, dst, wsem)

            @pl.when(jnp.logical_and(blen > qs, blen < qs + Q))
            def _(qs=qs, blen=blen, obase=obase):
                def row_copy(j, carry2):
                    jj = jnp.where(qs + j < blen, qs + j + 1, 0)
                    src_off = pl.multiple_of(jj * D_MODEL, D_MODEL)
                    dst_off = pl.multiple_of(
                        obase + (qs + j) * D_MODEL, D_MODEL)
                    pltpu.async_copy(
                        table_f.at[pl.ds(src_off, D_MODEL)],
                        pos_out.at[pl.ds(dst_off, D_MODEL)], wsem)
                    return carry2

                lax.fori_loop(0, Q, row_copy, 0)
        return carry

    lax.fori_loop(0, SB, do_batch, 0)

    for h in range(B_PER_W // LANES):
        lens16 = len_v[pl.ds(h * LANES, LANES)]
        for t in range(LANES // 2):
            len0 = lens16[2 * t]
            len1 = lens16[2 * t + 1]
            pair_base = (h * (LANES // 2) + t) * 2 * MAX_LEN

            def compute_idx(q, carry, len0=len0, len1=len1,
                            pair_base=pair_base):
                r_pair = _full(q * LANES) + lanes
                in_b1 = r_pair >= _full(MAX_LEN)
                pos = jnp.where(in_b1, r_pair - _full(MAX_LEN - 1),
                                r_pair + _full(1))
                lens = jnp.where(in_b1, _full(len1), _full(len0))
                idx = jnp.where(pos <= lens, pos, _full(0))
                idx_v[pl.ds(pair_base + q * LANES, LANES)] = idx
                return carry

            lax.fori_loop(0, GROUPS_PER_PAIR, compute_idx, 0)

    pltpu.async_copy(idx_v, idx_out.at[pl.ds(base_r, ROWS_PER_W)], isem)

    def drain(b, carry):
        pltpu.make_async_copy(
            pos_out.at[pl.ds(0, MAX_LEN * D_MODEL)],
            table_f.at[pl.ds(0, MAX_LEN * D_MODEL)],
            wsem).wait()
        return carry

    lax.fori_loop(0, SB, drain, 0)

    pltpu.make_async_copy(idx_v, idx_out.at[pl.ds(base_r, ROWS_PER_W)],
                          isem).wait()


def _pe_fill_tc_body(len_ref, pe_ref, pos_in_ref, out_ref):
    del pos_in_ref
    pid = pl.program_id(0)
    lens = len_ref[pl.ds(pid * BB, BB)]
    j = lax.broadcasted_iota(jnp.int32, (BB, MAX_LEN, 1), 1)
    mask = j < lens[:, None, None]
    out_ref[...] = jnp.where(mask, pe_ref[...][None, :, :],
                             jnp.float32(0.0))


_pe_fill_tc = pl.pallas_call(
    _pe_fill_tc_body,
    grid=(B_TC // BB,),
    in_specs=[
        pl.BlockSpec((BATCH,), lambda i: (0,)),
        pl.BlockSpec((MAX_LEN, D_MODEL), lambda i: (0, 0)),
        pl.BlockSpec(memory_space=pl.ANY),
    ],
    out_specs=pl.BlockSpec((BB, MAX_LEN, D_MODEL), lambda i: (i, 0, 0)),
    out_shape=jax.ShapeDtypeStruct((BATCH, MAX_LEN, D_MODEL), jnp.float32),
    input_output_aliases={2: 0},
)


def kernel(input_len, position_encoding):
    len_i32 = input_len.astype(jnp.int32)
    pe_flat = position_encoding.reshape(-1)
    pos_flat, idx_flat = _pe_fill_sc(pe_flat, len_i32)
    pos_3d = pos_flat.reshape(BATCH, MAX_LEN, D_MODEL)
    positions = _pe_fill_tc(len_i32, position_encoding[1:], pos_3d)
    return (positions, idx_flat.reshape(BATCH, MAX_LEN))
